# baseline (speedup 1.0000x reference)
import functools

import jax
import jax.numpy as jnp
from jax import lax
from jax.experimental import pallas as pl
from jax.experimental.pallas import tpu as pltpu
from jax.experimental.pallas import tpu_sc as plsc

N = 10000
E = 320000
D_IN = 128
HID = 64
D_OUT = 128
NG = 16

NC = 2
NS = 16
NW = NC * NS
CHUNK = 128
SLOW_CID = 0
NCH_SLOW = 70
NCH_FAST = 88
NCHMAX = NCH_FAST
EPAD = NS * (NCH_SLOW + NCH_FAST) * CHUNK
N_ACC = N + 112
RPS = N_ACC // NS

_MESH = plsc.VectorSubcoreMesh(core_axis_name="c", subcore_axis_name="s")


def _deg_body(dst_hbm, ones_hbm, zeros_hbm, out_hbm, dst_v, ones_v, sem, acc_sh):
    cid = lax.axis_index("c")
    sid = lax.axis_index("s")
    wid = sid * NC + cid
    nch = jnp.where(cid == SLOW_CID, NCH_SLOW, NCH_FAST)
    row = pl.ds(sid * RPS, RPS)
    pltpu.sync_copy(zeros_hbm.at[row], acc_sh.at[row])
    pltpu.sync_copy(ones_hbm, ones_v)
    pltpu.async_copy(dst_hbm.at[wid], dst_v, sem).wait()
    plsc.subcore_barrier()

    def body(j, c):
        pltpu.sync_copy(ones_v, acc_sh.at[dst_v.at[j]], add=True)
        return c

    lax.fori_loop(0, nch, body, 0, unroll=False)
    plsc.subcore_barrier()
    pltpu.sync_copy(acc_sh.at[row], out_hbm.at[cid, row])


_deg_pass = functools.partial(
    pl.kernel,
    compiler_params=pltpu.CompilerParams(use_tc_tiling_on_sc=False),
    out_type=jax.ShapeDtypeStruct((NC, N_ACC, 16), jnp.float32),
    mesh=_MESH,
    scratch_types=[
        pltpu.VMEM((NCHMAX, CHUNK), jnp.int32),
        pltpu.VMEM((CHUNK, 16), jnp.float32),
        pltpu.SemaphoreType.DMA,
        pltpu.VMEM_SHARED((N_ACC, 16), jnp.float32),
    ],
)(_deg_body)


def _edge_body(u_hbm, src_hbm, dst_hbm, zeros_hbm, out_hbm,
               src_v, dst_v, rows, gsems, sem_i, acc_sh):
    cid = lax.axis_index("c")
    sid = lax.axis_index("s")
    wid = sid * NC + cid
    nch = jnp.where(cid == SLOW_CID, NCH_SLOW, NCH_FAST)
    row = pl.ds(sid * RPS, RPS)
    pltpu.sync_copy(zeros_hbm.at[row], acc_sh.at[row])
    pltpu.async_copy(src_hbm.at[wid], src_v, sem_i).wait()
    pltpu.async_copy(dst_hbm.at[wid], dst_v, sem_i).wait()
    plsc.subcore_barrier()

    pltpu.async_copy(u_hbm.at[src_v.at[0]], rows[0], gsems[0])

    def body(j, c):
        for b in range(2):
            on = lax.rem(j, 2) == b

            @pl.when(on)
            def _():
                pltpu.make_async_copy(u_hbm.at[src_v.at[j]], rows[b], gsems[b]).wait()

                @pl.when(j + 1 < nch)
                def _():
                    bb = (b + 1) % 2
                    pltpu.async_copy(u_hbm.at[src_v.at[j + 1]], rows[bb], gsems[bb])
                pltpu.sync_copy(rows[b], acc_sh.at[dst_v.at[j]], add=True)
        return c

    lax.fori_loop(0, nch, body, 0, unroll=False)
    plsc.subcore_barrier()
    pltpu.sync_copy(acc_sh.at[row], out_hbm.at[cid, row])


_edge_pass = functools.partial(
    pl.kernel,
    compiler_params=pltpu.CompilerParams(use_tc_tiling_on_sc=False),
    out_type=jax.ShapeDtypeStruct((NC, N_ACC, HID), jnp.bfloat16),
    mesh=_MESH,
    scratch_types=[
        pltpu.VMEM((NCHMAX, CHUNK), jnp.int32),
        pltpu.VMEM((NCHMAX, CHUNK), jnp.int32),
        [pltpu.VMEM((CHUNK, HID), jnp.bfloat16)] * 2,
        [pltpu.SemaphoreType.DMA] * 2,
        pltpu.SemaphoreType.DMA,
        pltpu.VMEM_SHARED((N_ACC, HID), jnp.bfloat16),
    ],
)(_edge_body)


def _dis(degp):
    deg = degp[0, :, 0:1] + degp[1, :, 0:1] + 1.0
    return lax.rsqrt(deg)[:N, :]


def _stage_a(degp_ref, x_ref, w1_ref, u1_ref):
    h = jnp.dot(x_ref[...], w1_ref[...], preferred_element_type=jnp.float32)
    u1_ref[...] = (h * _dis(degp_ref[...])).astype(jnp.bfloat16)


def _stage_mid(degp_ref, accp_ref, u_ref, b_ref, w_ref, u2_ref):
    dis = _dis(degp_ref[...])
    acc = (accp_ref[0, :N, :].astype(jnp.float32)
           + accp_ref[1, :N, :].astype(jnp.float32)
           + u_ref[...].astype(jnp.float32))
    h = jnp.maximum(dis * acc + b_ref[...], 0.0)
    u2_ref[...] = (jnp.dot(h, w_ref[...], preferred_element_type=jnp.float32)
                   * dis).astype(jnp.bfloat16)


def _stage_out(degp_ref, accp_ref, u_ref, b_ref, batch_ref, wl_ref, bl_ref, z_ref):
    dis = _dis(degp_ref[...])
    acc = (accp_ref[0, :N, :].astype(jnp.float32)
           + accp_ref[1, :N, :].astype(jnp.float32)
           + u_ref[...].astype(jnp.float32))
    h = jnp.maximum(dis * acc + b_ref[...], 0.0)
    gid = lax.broadcasted_iota(jnp.int32, (NG, N), 0)
    sel = (jnp.broadcast_to(batch_ref[...], (NG, N)) == gid).astype(jnp.float32)
    sums = jnp.dot(sel, h, preferred_element_type=jnp.float32)
    counts = jnp.sum(sel, axis=1, keepdims=True)
    g = sums / jnp.maximum(counts, 1.0)
    z_ref[...] = jnp.dot(g, wl_ref[...], preferred_element_type=jnp.float32) + bl_ref[...]


def kernel(x, edge_index, batch, W1, b1, W2, b2, Wl, bl):
    src = edge_index[0]
    dst = edge_index[1]
    src_p = jnp.concatenate([src, jnp.zeros((EPAD - E,), jnp.int32)])
    dst_p = jnp.concatenate([dst, jnp.full((EPAD - E,), N, jnp.int32)])

    def _layout(flat, pad_val):
        rows_, off = [], 0
        for w in range(NW):
            n_w = (NCH_SLOW if (w % NC) == SLOW_CID else NCH_FAST) * CHUNK
            blk = flat[off:off + n_w]
            off += n_w
            if n_w < NCHMAX * CHUNK:
                blk = jnp.concatenate(
                    [blk, jnp.full((NCHMAX * CHUNK - n_w,), pad_val, jnp.int32)])
            rows_.append(blk.reshape(NCHMAX, CHUNK))
        return jnp.stack(rows_)

    srcp = _layout(src_p, 0)
    dstp = _layout(dst_p, N)
    ones16 = jnp.ones((CHUNK, 16), jnp.float32)
    zeros16 = jnp.zeros((N_ACC, 16), jnp.float32)
    zeros64 = jnp.zeros((N_ACC, HID), jnp.bfloat16)

    degp = _deg_pass(dstp, ones16, zeros16)

    u1 = pl.pallas_call(
        _stage_a,
        out_shape=jax.ShapeDtypeStruct((N, HID), jnp.bfloat16),
    )(degp, x, W1)

    acc1 = _edge_pass(u1, srcp, dstp, zeros64)

    u2 = pl.pallas_call(
        _stage_mid,
        out_shape=jax.ShapeDtypeStruct((N, HID), jnp.bfloat16),
    )(degp, acc1, u1, b1.reshape(1, HID), W2)

    acc2 = _edge_pass(u2, srcp, dstp, zeros64)

    z = pl.pallas_call(
        _stage_out,
        out_shape=jax.ShapeDtypeStruct((NG, D_OUT), jnp.float32),
    )(degp, acc2, u2, b2.reshape(1, HID), batch.reshape(1, N), Wl, bl.reshape(1, D_OUT))
    return z


# speedup vs baseline: 1.0838x; 1.0838x over previous
import functools

import jax
import jax.numpy as jnp
from jax import lax
from jax.experimental import pallas as pl
from jax.experimental.pallas import tpu as pltpu
from jax.experimental.pallas import tpu_sc as plsc

N = 10000
E = 320000
D_IN = 128
HID = 64
D_OUT = 128
NG = 16

NC = 2
NS = 16
NW = NC * NS
CHUNK = 128
NCH = -(-E // (NW * CHUNK))
EPAD = NW * NCH * CHUNK
N_ACC = N + 112
RPS = N_ACC // NS

_MESH = plsc.VectorSubcoreMesh(core_axis_name="c", subcore_axis_name="s")


def _deg_body(dst_hbm, ones_hbm, zeros_hbm, out_hbm, dst_v, ones_v, sem, acc_sh):
    cid = lax.axis_index("c")
    sid = lax.axis_index("s")
    wid = sid * NC + cid
    row = pl.ds(sid * RPS, RPS)
    pltpu.sync_copy(zeros_hbm.at[row], acc_sh.at[row])
    pltpu.sync_copy(ones_hbm, ones_v)
    pltpu.async_copy(dst_hbm.at[wid], dst_v, sem).wait()
    plsc.subcore_barrier()

    def body(j, c):
        pltpu.sync_copy(ones_v, acc_sh.at[dst_v.at[j]], add=True)
        return c

    lax.fori_loop(0, NCH, body, 0, unroll=False)
    plsc.subcore_barrier()
    pltpu.sync_copy(acc_sh.at[row], out_hbm.at[cid, row])


_deg_pass = functools.partial(
    pl.kernel,
    compiler_params=pltpu.CompilerParams(use_tc_tiling_on_sc=False),
    out_type=jax.ShapeDtypeStruct((NC, N_ACC, 16), jnp.float32),
    mesh=_MESH,
    scratch_types=[
        pltpu.VMEM((NCH, CHUNK), jnp.int32),
        pltpu.VMEM((CHUNK, 16), jnp.float32),
        pltpu.SemaphoreType.DMA,
        pltpu.VMEM_SHARED((N_ACC, 16), jnp.float32),
    ],
)(_deg_body)


def _edge_body(u_hbm, src_hbm, dst_hbm, zeros_hbm, out_hbm,
               src_v, dst_v, rows, gsems, sem_i, acc_sh):
    cid = lax.axis_index("c")
    sid = lax.axis_index("s")
    wid = sid * NC + cid
    row = pl.ds(sid * RPS, RPS)
    pltpu.sync_copy(zeros_hbm.at[row], acc_sh.at[row])
    pltpu.async_copy(src_hbm.at[wid], src_v, sem_i).wait()
    pltpu.async_copy(dst_hbm.at[wid], dst_v, sem_i).wait()
    plsc.subcore_barrier()

    pltpu.async_copy(u_hbm.at[src_v.at[0]], rows[0], gsems[0])

    def body(j, c):
        for b in range(2):
            on = lax.rem(j, 2) == b

            @pl.when(on)
            def _():
                pltpu.make_async_copy(u_hbm.at[src_v.at[j]], rows[b], gsems[b]).wait()

                @pl.when(j + 1 < NCH)
                def _():
                    bb = (b + 1) % 2
                    pltpu.async_copy(u_hbm.at[src_v.at[j + 1]], rows[bb], gsems[bb])
                pltpu.sync_copy(rows[b], acc_sh.at[dst_v.at[j]], add=True)
        return c

    lax.fori_loop(0, NCH, body, 0, unroll=False)
    plsc.subcore_barrier()
    pltpu.sync_copy(acc_sh.at[row], out_hbm.at[cid, row])


_edge_pass = functools.partial(
    pl.kernel,
    compiler_params=pltpu.CompilerParams(use_tc_tiling_on_sc=False),
    out_type=jax.ShapeDtypeStruct((NC, N_ACC, HID), jnp.bfloat16),
    mesh=_MESH,
    scratch_types=[
        pltpu.VMEM((NCH, CHUNK), jnp.int32),
        pltpu.VMEM((NCH, CHUNK), jnp.int32),
        [pltpu.VMEM((CHUNK, HID), jnp.bfloat16)] * 2,
        [pltpu.SemaphoreType.DMA] * 2,
        pltpu.SemaphoreType.DMA,
        pltpu.VMEM_SHARED((N_ACC, HID), jnp.bfloat16),
    ],
)(_edge_body)


def _dis(degp):
    deg = degp[0, :, 0:1] + degp[1, :, 0:1] + 1.0
    return lax.rsqrt(deg)[:N, :]


def _stage_a(degp_ref, x_ref, w1_ref, u1_ref):
    h = jnp.dot(x_ref[...], w1_ref[...], preferred_element_type=jnp.float32)
    u1_ref[...] = (h * _dis(degp_ref[...])).astype(jnp.bfloat16)


def _stage_mid(degp_ref, accp_ref, u_ref, b_ref, w_ref, u2_ref):
    dis = _dis(degp_ref[...])
    acc = (accp_ref[0, :N, :].astype(jnp.float32)
           + accp_ref[1, :N, :].astype(jnp.float32)
           + u_ref[...].astype(jnp.float32))
    h = jnp.maximum(dis * acc + b_ref[...], 0.0)
    u2_ref[...] = (jnp.dot(h, w_ref[...], preferred_element_type=jnp.float32)
                   * dis).astype(jnp.bfloat16)


def _stage_out(degp_ref, accp_ref, u_ref, b_ref, batch_ref, wl_ref, bl_ref, z_ref):
    dis = _dis(degp_ref[...])
    acc = (accp_ref[0, :N, :].astype(jnp.float32)
           + accp_ref[1, :N, :].astype(jnp.float32)
           + u_ref[...].astype(jnp.float32))
    h = jnp.maximum(dis * acc + b_ref[...], 0.0)
    gid = lax.broadcasted_iota(jnp.int32, (NG, N), 0)
    sel = (jnp.broadcast_to(batch_ref[...], (NG, N)) == gid).astype(jnp.float32)
    sums = jnp.dot(sel, h, preferred_element_type=jnp.float32)
    counts = jnp.sum(sel, axis=1, keepdims=True)
    g = sums / jnp.maximum(counts, 1.0)
    z_ref[...] = jnp.dot(g, wl_ref[...], preferred_element_type=jnp.float32) + bl_ref[...]


def kernel(x, edge_index, batch, W1, b1, W2, b2, Wl, bl):
    src = edge_index[0]
    dst = edge_index[1]
    fill = EPAD - E
    srcp = jnp.concatenate([src, jnp.zeros((fill,), jnp.int32)]).reshape(NW, NCH, CHUNK)
    dstp = jnp.concatenate([dst, jnp.full((fill,), N, jnp.int32)]).reshape(NW, NCH, CHUNK)
    ones16 = jnp.ones((CHUNK, 16), jnp.float32)
    zeros16 = jnp.zeros((N_ACC, 16), jnp.float32)
    zeros64 = jnp.zeros((N_ACC, HID), jnp.bfloat16)

    degp = _deg_pass(dstp, ones16, zeros16)

    u1 = pl.pallas_call(
        _stage_a,
        out_shape=jax.ShapeDtypeStruct((N, HID), jnp.bfloat16),
    )(degp, x, W1)

    acc1 = _edge_pass(u1, srcp, dstp, zeros64)

    u2 = pl.pallas_call(
        _stage_mid,
        out_shape=jax.ShapeDtypeStruct((N, HID), jnp.bfloat16),
    )(degp, acc1, u1, b1.reshape(1, HID), W2)

    acc2 = _edge_pass(u2, srcp, dstp, zeros64)

    z = pl.pallas_call(
        _stage_out,
        out_shape=jax.ShapeDtypeStruct((NG, D_OUT), jnp.float32),
    )(degp, acc2, u2, b2.reshape(1, HID), batch.reshape(1, N), Wl, bl.reshape(1, D_OUT))
    return z
